# baseline (speedup 1.0000x reference)
import functools

import jax
import jax.numpy as jnp
from jax import lax
from jax.experimental import pallas as pl
from jax.experimental.pallas import tpu as pltpu
from jax.experimental.pallas import tpu_sc as plsc

B = 1024
MAX_LEN = 20
VOCAB = 100000
EMB_DIM = 32
HID = MAX_LEN * EMB_DIM
N_TOK = B * MAX_LEN

NC, NS = 2, 16
NW = NC * NS
CHUNK = 128
TOK_PER_W = N_TOK // NW
CHUNKS_PER_W = TOK_PER_W // CHUNK


@functools.cache
def _make_sc_gather():
    mesh = plsc.VectorSubcoreMesh(
        core_axis_name="c", subcore_axis_name="s", num_cores=NC, num_subcores=NS
    )

    @functools.partial(
        pl.kernel,
        out_type=jax.ShapeDtypeStruct((N_TOK, EMB_DIM), jnp.float32),
        mesh=mesh,
        scratch_types=[
            pltpu.VMEM((CHUNKS_PER_W, CHUNK), jnp.int32),
            pltpu.VMEM((TOK_PER_W, EMB_DIM), jnp.float32),
            pltpu.SemaphoreType.DMA,
        ],
        compiler_params=pltpu.CompilerParams(use_tc_tiling_on_sc=False),
    )
    def _sc_gather(idx_hbm, table_hbm, out_hbm, idx_v, rows_v, sem):
        wid = lax.axis_index("s") * NC + lax.axis_index("c")
        pltpu.sync_copy(idx_hbm.at[wid], idx_v)
        copies = [
            pltpu.async_copy(
                table_hbm.at[idx_v.at[j]],
                rows_v.at[pl.ds(j * CHUNK, CHUNK)],
                sem,
            )
            for j in range(CHUNKS_PER_W)
        ]
        for c in copies:
            c.wait()
        pltpu.sync_copy(rows_v, out_hbm.at[pl.ds(wid * TOK_PER_W, TOK_PER_W)])

    return _sc_gather


VB = 2048
_NFULL = VOCAB // VB
_TAIL = VOCAB - _NFULL * VB
_TAIL_OFF = _NFULL * VB
_NSTEP = _NFULL + 1


def _mm_body(h_hbm, w_hbm, b_hbm, o_hbm,
             h_vmem, b_vmem, w_buf, o_buf, w_tail, o_tail,
             ld_sem, w_sems, o_sems, wt_sem, ot_sem):
    i = pl.program_id(0)
    slot = lax.rem(i, 2)
    nxt = lax.rem(i + 1, 2)

    @pl.when(i == 0)
    def _prologue():
        pltpu.make_async_copy(h_hbm, h_vmem, ld_sem).start()
        pltpu.make_async_copy(b_hbm, b_vmem, ld_sem).start()
        pltpu.make_async_copy(
            w_hbm.at[pl.ds(0, VB)], w_buf.at[0], w_sems.at[0]).start()
        pltpu.make_async_copy(h_hbm, h_vmem, ld_sem).wait()
        pltpu.make_async_copy(b_hbm, b_vmem, ld_sem).wait()

    @pl.when(i + 1 < _NFULL)
    def _prefetch():
        pltpu.make_async_copy(
            w_hbm.at[pl.ds((i + 1) * VB, VB)], w_buf.at[nxt],
            w_sems.at[nxt]).start()

    @pl.when(i + 1 == _NFULL)
    def _prefetch_tail():
        pltpu.make_async_copy(
            w_hbm.at[pl.ds(_TAIL_OFF, _TAIL)], w_tail, wt_sem).start()

    @pl.when(i < _NFULL)
    def _full_step():
        pltpu.make_async_copy(
            w_hbm.at[pl.ds(i * VB, VB)], w_buf.at[slot],
            w_sems.at[slot]).wait()

        @pl.when(i >= 2)
        def _drain_prev():
            pltpu.make_async_copy(
                o_buf.at[slot], o_hbm.at[:, pl.ds((i - 2) * VB, VB)],
                o_sems.at[slot]).wait()

        o_buf[slot] = lax.dot_general(
            h_vmem[...], w_buf[slot],
            dimension_numbers=(((1,), (1,)), ((), ())),
            preferred_element_type=jnp.float32,
        ) + b_vmem[:, pl.ds(i * VB, VB)]

        pltpu.make_async_copy(
            o_buf.at[slot], o_hbm.at[:, pl.ds(i * VB, VB)],
            o_sems.at[slot]).start()

    @pl.when(i == _NFULL)
    def _tail_step():
        pltpu.make_async_copy(
            w_hbm.at[pl.ds(_TAIL_OFF, _TAIL)], w_tail, wt_sem).wait()

        o_tail[...] = lax.dot_general(
            h_vmem[...], w_tail[...],
            dimension_numbers=(((1,), (1,)), ((), ())),
            preferred_element_type=jnp.float32,
        ) + b_vmem[:, pl.ds(_TAIL_OFF, _TAIL)]

        pltpu.make_async_copy(
            o_tail, o_hbm.at[:, pl.ds(_TAIL_OFF, _TAIL)], ot_sem).start()

        pltpu.make_async_copy(
            o_buf.at[0], o_hbm.at[:, pl.ds((_NFULL - 2) * VB, VB)],
            o_sems.at[0]).wait()
        pltpu.make_async_copy(
            o_buf.at[1], o_hbm.at[:, pl.ds((_NFULL - 1) * VB, VB)],
            o_sems.at[1]).wait()
        pltpu.make_async_copy(
            o_tail, o_hbm.at[:, pl.ds(_TAIL_OFF, _TAIL)], ot_sem).wait()


def _tc_matmul(h, W, b2d):
    return pl.pallas_call(
        _mm_body,
        grid=(_NSTEP,),
        in_specs=[
            pl.BlockSpec(memory_space=pltpu.HBM),
            pl.BlockSpec(memory_space=pltpu.HBM),
            pl.BlockSpec(memory_space=pltpu.HBM),
        ],
        out_specs=pl.BlockSpec(memory_space=pltpu.HBM),
        out_shape=jax.ShapeDtypeStruct((B, VOCAB), jnp.float32),
        scratch_shapes=[
            pltpu.VMEM((B, HID), jnp.float32),
            pltpu.VMEM((1, VOCAB), jnp.float32),
            pltpu.VMEM((2, VB, HID), jnp.float32),
            pltpu.VMEM((2, B, VB), jnp.float32),
            pltpu.VMEM((_TAIL, HID), jnp.float32),
            pltpu.VMEM((B, _TAIL), jnp.float32),
            pltpu.SemaphoreType.DMA,
            pltpu.SemaphoreType.DMA((2,)),
            pltpu.SemaphoreType.DMA((2,)),
            pltpu.SemaphoreType.DMA,
            pltpu.SemaphoreType.DMA,
        ],
        compiler_params=pltpu.CompilerParams(
            dimension_semantics=("arbitrary",),
        ),
    )(h, W, b2d)


def kernel(x, emb, W, b):
    idx = x.astype(jnp.int32).reshape(NW, CHUNKS_PER_W, CHUNK)
    h = _make_sc_gather()(idx, emb)
    h = h.reshape(B, HID)
    return _tc_matmul(h, W, b.reshape(1, VOCAB))


# speedup vs baseline: 2.1512x; 2.1512x over previous
import functools

import jax
import jax.numpy as jnp
from jax import lax
from jax.experimental import pallas as pl
from jax.experimental.pallas import tpu as pltpu
from jax.experimental.pallas import tpu_sc as plsc

B = 1024
MAX_LEN = 20
VOCAB = 100000
EMB_DIM = 32
HID = MAX_LEN * EMB_DIM
N_TOK = B * MAX_LEN

NC, NS = 2, 16
NW = NC * NS
CHUNK = 128
TOK_PER_W = N_TOK // NW
CHUNKS_PER_W = TOK_PER_W // CHUNK


@functools.cache
def _make_sc_gather():
    mesh = plsc.VectorSubcoreMesh(
        core_axis_name="c", subcore_axis_name="s", num_cores=NC, num_subcores=NS
    )

    @functools.partial(
        pl.kernel,
        out_type=jax.ShapeDtypeStruct((N_TOK, EMB_DIM), jnp.float32),
        mesh=mesh,
        scratch_types=[
            pltpu.VMEM((CHUNKS_PER_W, CHUNK), jnp.int32),
            pltpu.VMEM((TOK_PER_W, EMB_DIM), jnp.float32),
            pltpu.SemaphoreType.DMA,
        ],
        compiler_params=pltpu.CompilerParams(use_tc_tiling_on_sc=False),
    )
    def _sc_gather(idx_hbm, table_hbm, out_hbm, idx_v, rows_v, sem):
        wid = lax.axis_index("s") * NC + lax.axis_index("c")
        pltpu.sync_copy(idx_hbm.at[wid], idx_v)
        copies = [
            pltpu.async_copy(
                table_hbm.at[idx_v.at[j]],
                rows_v.at[pl.ds(j * CHUNK, CHUNK)],
                sem,
            )
            for j in range(CHUNKS_PER_W)
        ]
        for c in copies:
            c.wait()
        pltpu.sync_copy(rows_v, out_hbm.at[pl.ds(wid * TOK_PER_W, TOK_PER_W)])

    return _sc_gather


VB = 2000
_NSTEP = VOCAB // VB


def _mm_body(h_ref, w_ref, b_ref, o_ref):
    i = pl.program_id(0)
    onehot = (lax.broadcasted_iota(jnp.int32, (1, 128), 1) == i).astype(
        jnp.float32)
    b_col = jnp.sum(b_ref[...] * onehot, axis=1, keepdims=True)
    o_ref[...] = lax.dot_general(
        w_ref[...], h_ref[...],
        dimension_numbers=(((1,), (1,)), ((), ())),
        preferred_element_type=jnp.float32,
    ) + b_col


def _tc_matmul(h, W, bL):
    return pl.pallas_call(
        _mm_body,
        grid=(_NSTEP,),
        in_specs=[
            pl.BlockSpec((B, HID), lambda v: (0, 0)),
            pl.BlockSpec((VB, HID), lambda v: (v, 0)),
            pl.BlockSpec((VB, 128), lambda v: (0, 0)),
        ],
        out_specs=pl.BlockSpec((VB, B), lambda v: (v, 0)),
        out_shape=jax.ShapeDtypeStruct((VOCAB, B), jnp.float32),
        compiler_params=pltpu.CompilerParams(
            dimension_semantics=("arbitrary",),
        ),
    )(h, W, bL)


def kernel(x, emb, W, b):
    idx = x.astype(jnp.int32).reshape(NW, CHUNKS_PER_W, CHUNK)
    h = _make_sc_gather()(idx, emb)
    h = h.reshape(B, HID)
    bL = jnp.zeros((VB, 128), jnp.float32).at[:, :_NSTEP].set(
        b.reshape(_NSTEP, VB).T)
    outT = _tc_matmul(h, W, bL)
    return outT.T


# speedup vs baseline: 2.1717x; 1.0095x over previous
import functools

import jax
import jax.numpy as jnp
from jax import lax
from jax.experimental import pallas as pl
from jax.experimental.pallas import tpu as pltpu
from jax.experimental.pallas import tpu_sc as plsc

B = 1024
MAX_LEN = 20
VOCAB = 100000
EMB_DIM = 32
HID = MAX_LEN * EMB_DIM
N_TOK = B * MAX_LEN

NC, NS = 2, 16
NW = NC * NS
CHUNK = 128
TOK_PER_W = N_TOK // NW
CHUNKS_PER_W = TOK_PER_W // CHUNK


@functools.cache
def _make_sc_gather():
    mesh = plsc.VectorSubcoreMesh(
        core_axis_name="c", subcore_axis_name="s", num_cores=NC, num_subcores=NS
    )

    @functools.partial(
        pl.kernel,
        out_type=jax.ShapeDtypeStruct((N_TOK, EMB_DIM), jnp.float32),
        mesh=mesh,
        scratch_types=[
            pltpu.VMEM((CHUNKS_PER_W, CHUNK), jnp.int32),
            pltpu.VMEM((TOK_PER_W, EMB_DIM), jnp.float32),
            pltpu.SemaphoreType.DMA,
        ],
        compiler_params=pltpu.CompilerParams(use_tc_tiling_on_sc=False),
    )
    def _sc_gather(idx_hbm, table_hbm, out_hbm, idx_v, rows_v, sem):
        wid = lax.axis_index("s") * NC + lax.axis_index("c")
        pltpu.sync_copy(idx_hbm.at[wid], idx_v)
        copies = [
            pltpu.async_copy(
                table_hbm.at[idx_v.at[j]],
                rows_v.at[pl.ds(j * CHUNK, CHUNK)],
                sem,
            )
            for j in range(CHUNKS_PER_W)
        ]
        for c in copies:
            c.wait()
        pltpu.sync_copy(rows_v, out_hbm.at[pl.ds(wid * TOK_PER_W, TOK_PER_W)])

    return _sc_gather


VB = 2000
_NSTEP = VOCAB // VB


def _mm_body(h_hbm, w_hbm, b_hbm, o_hbm,
             h_vmem, b_vmem, w_buf, o_buf, ld_sem, w_sems, o_sems):
    i = pl.program_id(0)
    slot = lax.rem(i, 2)
    nxt = lax.rem(i + 1, 2)

    @pl.when(i == 0)
    def _prologue():
        pltpu.make_async_copy(h_hbm, h_vmem, ld_sem).start()
        pltpu.make_async_copy(b_hbm, b_vmem, ld_sem).start()
        pltpu.make_async_copy(
            w_hbm.at[pl.ds(0, VB)], w_buf.at[0], w_sems.at[0]).start()
        pltpu.make_async_copy(h_hbm, h_vmem, ld_sem).wait()
        pltpu.make_async_copy(b_hbm, b_vmem, ld_sem).wait()

    @pl.when(i + 1 < _NSTEP)
    def _prefetch():
        pltpu.make_async_copy(
            w_hbm.at[pl.ds((i + 1) * VB, VB)], w_buf.at[nxt],
            w_sems.at[nxt]).start()

    pltpu.make_async_copy(
        w_hbm.at[pl.ds(i * VB, VB)], w_buf.at[slot], w_sems.at[slot]).wait()

    @pl.when(i >= 2)
    def _drain_prev():
        pltpu.make_async_copy(
            o_buf.at[slot], o_hbm.at[pl.ds((i - 2) * VB, VB)],
            o_sems.at[slot]).wait()

    onehot = (lax.broadcasted_iota(jnp.int32, (1, 128), 1) == i).astype(
        jnp.float32)
    b_col = jnp.sum(b_vmem[...] * onehot, axis=1, keepdims=True)
    o_buf[slot] = lax.dot_general(
        w_buf[slot], h_vmem[...],
        dimension_numbers=(((1,), (1,)), ((), ())),
        preferred_element_type=jnp.float32,
    ) + b_col

    pltpu.make_async_copy(
        o_buf.at[slot], o_hbm.at[pl.ds(i * VB, VB)], o_sems.at[slot]).start()

    @pl.when(i == _NSTEP - 1)
    def _epilogue():
        pltpu.make_async_copy(
            o_buf.at[nxt], o_hbm.at[pl.ds((i - 1) * VB, VB)],
            o_sems.at[nxt]).wait()
        pltpu.make_async_copy(
            o_buf.at[slot], o_hbm.at[pl.ds(i * VB, VB)],
            o_sems.at[slot]).wait()


def _tc_matmul(h, W, bL):
    return pl.pallas_call(
        _mm_body,
        grid=(_NSTEP,),
        in_specs=[
            pl.BlockSpec(memory_space=pltpu.HBM),
            pl.BlockSpec(memory_space=pltpu.HBM),
            pl.BlockSpec(memory_space=pltpu.HBM),
        ],
        out_specs=pl.BlockSpec(memory_space=pltpu.HBM),
        out_shape=jax.ShapeDtypeStruct((VOCAB, B), jnp.float32),
        scratch_shapes=[
            pltpu.VMEM((B, HID), jnp.float32),
            pltpu.VMEM((VB, 128), jnp.float32),
            pltpu.VMEM((2, VB, HID), jnp.float32),
            pltpu.VMEM((2, VB, B), jnp.float32),
            pltpu.SemaphoreType.DMA,
            pltpu.SemaphoreType.DMA((2,)),
            pltpu.SemaphoreType.DMA((2,)),
        ],
        compiler_params=pltpu.CompilerParams(
            dimension_semantics=("arbitrary",),
        ),
    )(h, W, bL)


def kernel(x, emb, W, b):
    idx = x.astype(jnp.int32).reshape(NW, CHUNKS_PER_W, CHUNK)
    h = _make_sc_gather()(idx, emb)
    h = h.reshape(B, HID)
    bL = jnp.zeros((VB, 128), jnp.float32).at[:, :_NSTEP].set(
        b.reshape(_NSTEP, VB).T)
    outT = _tc_matmul(h, W, bL)
    return outT.T


# speedup vs baseline: 2.2092x; 1.0173x over previous
import functools

import jax
import jax.numpy as jnp
from jax import lax
from jax.experimental import pallas as pl
from jax.experimental.pallas import tpu as pltpu
from jax.experimental.pallas import tpu_sc as plsc

B = 1024
MAX_LEN = 20
VOCAB = 100000
EMB_DIM = 32
HID = MAX_LEN * EMB_DIM
N_TOK = B * MAX_LEN

NC, NS = 2, 16
NW = NC * NS
CHUNK = 128
TOK_PER_W = N_TOK // NW
CHUNKS_PER_W = TOK_PER_W // CHUNK


@functools.cache
def _make_sc_gather():
    mesh = plsc.VectorSubcoreMesh(
        core_axis_name="c", subcore_axis_name="s", num_cores=NC, num_subcores=NS
    )

    @functools.partial(
        pl.kernel,
        out_type=jax.ShapeDtypeStruct((N_TOK, EMB_DIM), jnp.float32),
        mesh=mesh,
        scratch_types=[
            pltpu.VMEM((CHUNKS_PER_W, CHUNK), jnp.int32),
            pltpu.VMEM((TOK_PER_W, EMB_DIM), jnp.float32),
            pltpu.SemaphoreType.DMA,
        ],
        compiler_params=pltpu.CompilerParams(use_tc_tiling_on_sc=False),
    )
    def _sc_gather(idx_hbm, table_hbm, out_hbm, idx_v, rows_v, sem):
        wid = lax.axis_index("s") * NC + lax.axis_index("c")
        pltpu.sync_copy(idx_hbm.at[wid], idx_v)
        copies = [
            pltpu.async_copy(
                table_hbm.at[idx_v.at[j]],
                rows_v.at[pl.ds(j * CHUNK, CHUNK)],
                sem,
            )
            for j in range(CHUNKS_PER_W)
        ]
        for c in copies:
            c.wait()
        pltpu.sync_copy(rows_v, out_hbm.at[pl.ds(wid * TOK_PER_W, TOK_PER_W)])

    return _sc_gather


VB = 4000
_NSTEP = VOCAB // VB


def _mm_body(h_hbm, w_hbm, b_hbm, o_hbm,
             h_vmem, b_vmem, w_buf, o_buf, ld_sem, w_sems, o_sems):
    i = pl.program_id(0)
    slot = lax.rem(i, 2)
    nxt = lax.rem(i + 1, 2)

    @pl.when(i == 0)
    def _prologue():
        pltpu.make_async_copy(h_hbm, h_vmem, ld_sem).start()
        pltpu.make_async_copy(b_hbm, b_vmem, ld_sem).start()
        pltpu.make_async_copy(
            w_hbm.at[pl.ds(0, VB)], w_buf.at[0], w_sems.at[0]).start()
        pltpu.make_async_copy(h_hbm, h_vmem, ld_sem).wait()
        pltpu.make_async_copy(b_hbm, b_vmem, ld_sem).wait()

    @pl.when(i + 1 < _NSTEP)
    def _prefetch():
        pltpu.make_async_copy(
            w_hbm.at[pl.ds((i + 1) * VB, VB)], w_buf.at[nxt],
            w_sems.at[nxt]).start()

    pltpu.make_async_copy(
        w_hbm.at[pl.ds(i * VB, VB)], w_buf.at[slot], w_sems.at[slot]).wait()

    @pl.when(i >= 2)
    def _drain_prev():
        pltpu.make_async_copy(
            o_buf.at[slot], o_hbm.at[pl.ds((i - 2) * VB, VB)],
            o_sems.at[slot]).wait()

    onehot = (lax.broadcasted_iota(jnp.int32, (1, 128), 1) == i).astype(
        jnp.float32)
    b_col = jnp.sum(b_vmem[...] * onehot, axis=1, keepdims=True)
    o_buf[slot] = lax.dot_general(
        w_buf[slot], h_vmem[...],
        dimension_numbers=(((1,), (1,)), ((), ())),
        preferred_element_type=jnp.float32,
    ) + b_col

    pltpu.make_async_copy(
        o_buf.at[slot], o_hbm.at[pl.ds(i * VB, VB)], o_sems.at[slot]).start()

    @pl.when(i == _NSTEP - 1)
    def _epilogue():
        pltpu.make_async_copy(
            o_buf.at[nxt], o_hbm.at[pl.ds((i - 1) * VB, VB)],
            o_sems.at[nxt]).wait()
        pltpu.make_async_copy(
            o_buf.at[slot], o_hbm.at[pl.ds(i * VB, VB)],
            o_sems.at[slot]).wait()


def _tc_matmul(h, W, bL):
    return pl.pallas_call(
        _mm_body,
        grid=(_NSTEP,),
        in_specs=[
            pl.BlockSpec(memory_space=pltpu.HBM),
            pl.BlockSpec(memory_space=pltpu.HBM),
            pl.BlockSpec(memory_space=pltpu.HBM),
        ],
        out_specs=pl.BlockSpec(memory_space=pltpu.HBM),
        out_shape=jax.ShapeDtypeStruct((VOCAB, B), jnp.float32),
        scratch_shapes=[
            pltpu.VMEM((B, HID), jnp.float32),
            pltpu.VMEM((VB, 128), jnp.float32),
            pltpu.VMEM((2, VB, HID), jnp.float32),
            pltpu.VMEM((2, VB, B), jnp.float32),
            pltpu.SemaphoreType.DMA,
            pltpu.SemaphoreType.DMA((2,)),
            pltpu.SemaphoreType.DMA((2,)),
        ],
        compiler_params=pltpu.CompilerParams(
            dimension_semantics=("arbitrary",),
            vmem_limit_bytes=100 * 1024 * 1024,
        ),
    )(h, W, bL)


def kernel(x, emb, W, b):
    idx = x.astype(jnp.int32).reshape(NW, CHUNKS_PER_W, CHUNK)
    h = _make_sc_gather()(idx, emb)
    h = h.reshape(B, HID)
    bL = jnp.zeros((VB, 128), jnp.float32).at[:, :_NSTEP].set(
        b.reshape(_NSTEP, VB).T)
    outT = _tc_matmul(h, W, bL)
    return outT.T
